# baseline (speedup 1.0000x reference)
import jax
import jax.numpy as jnp
from jax import lax
from jax.experimental import pallas as pl
from jax.experimental.pallas import tpu as pltpu
from jax.experimental.pallas import tpu_sc as plsc

B = 16384
ED = 64
OUT_D = 3 * ED
SV, VV, TV = 3, 4, 3
NROWS_T = SV * VV + TV * SV + VV * TV
REP = 32

NC, NS, L = 2, 16, 16
NW = NC * NS
BPW = B // NW
FPW = BPW * OUT_D // 128
MPW = BPW // 2
CHUNK = 128
NCHUNK = FPW // CHUNK


def _body(sess_hbm, vol_hbm, trend_hbm, tt_hbm, out_hbm,
          sidx_v, vidx_v, tidx_v, idx_v, rows_v, sem):
    wid = lax.axis_index("s") * NC + lax.axis_index("c")
    base = wid * BPW

    pltpu.sync_copy(sess_hbm.at[pl.ds(base, BPW)], sidx_v)
    pltpu.sync_copy(vol_hbm.at[pl.ds(base, BPW)], vidx_v)
    pltpu.sync_copy(trend_hbm.at[pl.ds(base, BPW)], tidx_v)

    lanes = lax.iota(jnp.int32, L)
    for j in range(MPW // L):
        even = 2 * (j * L + lanes)
        odd = even + 1
        s_e = jnp.minimum(jnp.maximum(plsc.load_gather(sidx_v, [even]), 0), SV - 1)
        v_e = jnp.minimum(jnp.maximum(plsc.load_gather(vidx_v, [even]), 0), VV - 1)
        t_e = jnp.minimum(jnp.maximum(plsc.load_gather(tidx_v, [even]), 0), TV - 1)
        s_o = jnp.minimum(jnp.maximum(plsc.load_gather(sidx_v, [odd]), 0), SV - 1)
        v_o = jnp.minimum(jnp.maximum(plsc.load_gather(vidx_v, [odd]), 0), VV - 1)
        t_o = jnp.minimum(jnp.maximum(plsc.load_gather(tidx_v, [odd]), 0), TV - 1)
        salt = NROWS_T * ((wid + j) % REP)
        i0 = s_e * VV + v_e + salt
        i1 = (SV * VV) + t_e * SV + s_o + salt
        i2 = (SV * VV + TV * SV) + v_o * TV + t_o + salt
        for k, ik in ((0, i0), (1, i1), (2, i2)):
            pos = 3 * (j * L + lanes) + k
            plsc.store_scatter(idx_v, [pos // CHUNK, pos % CHUNK], ik)

    copies = [
        pltpu.async_copy(
            tt_hbm.at[idx_v.at[q]],
            rows_v.at[pl.ds(q * CHUNK, CHUNK)],
            sem,
        )
        for q in range(NCHUNK)
    ]
    for c in copies:
        c.wait()

    pltpu.sync_copy(rows_v, out_hbm.at[pl.ds(wid * FPW, FPW)])


def kernel(session_id, vol_regime_id, trend_regime_id,
           session_table, vol_table, trend_table):
    c0 = jnp.arange(SV * VV, dtype=jnp.int32)
    c1 = jnp.arange(TV * SV, dtype=jnp.int32)
    c2 = jnp.arange(VV * TV, dtype=jnp.int32)
    t0 = jnp.concatenate(
        [jnp.take(session_table, c0 // VV, axis=0),
         jnp.take(vol_table, c0 % VV, axis=0)], axis=-1)
    t1 = jnp.concatenate(
        [jnp.take(trend_table, c1 // SV, axis=0),
         jnp.take(session_table, c1 % SV, axis=0)], axis=-1)
    t2 = jnp.concatenate(
        [jnp.take(vol_table, c2 // TV, axis=0),
         jnp.take(trend_table, c2 % TV, axis=0)], axis=-1)
    tt = jnp.tile(jnp.concatenate([t0, t1, t2], axis=0), (REP, 1))

    run = pl.kernel(
        _body,
        mesh=plsc.VectorSubcoreMesh(core_axis_name="c", subcore_axis_name="s"),
        out_type=jax.ShapeDtypeStruct((B * OUT_D // 128, 128), jnp.float32),
        scratch_types=[
            pltpu.VMEM((BPW,), jnp.int32),
            pltpu.VMEM((BPW,), jnp.int32),
            pltpu.VMEM((BPW,), jnp.int32),
            pltpu.VMEM((NCHUNK, CHUNK), jnp.int32),
            pltpu.VMEM((FPW, 128), jnp.float32),
            pltpu.SemaphoreType.DMA,
        ],
        compiler_params=pltpu.CompilerParams(
            use_tc_tiling_on_sc=False, needs_layout_passes=False),
    )
    flat = run(
        session_id.astype(jnp.int32),
        vol_regime_id.astype(jnp.int32),
        trend_regime_id.astype(jnp.int32),
        tt,
    )
    return flat.reshape(B, OUT_D)
